# Initial kernel scaffold; baseline (speedup 1.0000x reference)
import jax
import jax.numpy as jnp
from jax.experimental import pallas as pl


def kernel(input_ids, table):
    raise NotImplementedError("write your pallas kernel here")


# speedup vs baseline: 13.8341x; 13.8341x over previous
import functools

import jax
import jax.numpy as jnp
from jax import lax
from jax.experimental import pallas as pl
from jax.experimental.pallas import tpu as pltpu
from jax.experimental.pallas import tpu_sc as plsc

B = 4096
L = 200
D = 128
NC = 2
NS = 16
LANES = 16
NW = NC * NS
BPW = B // NW
DV = D // LANES
LEN_BLK = 512
CHUNKS = ((0, 128), (128, 72))


def _len_body(ids_ref, out_ref):
    lens = jnp.sum((ids_ref[...] != 0).astype(jnp.float32), axis=1, keepdims=True)
    out_ref[...] = jnp.broadcast_to(lens, (LEN_BLK, LANES))


def _lengths(input_ids):
    return pl.pallas_call(
        _len_body,
        out_shape=jax.ShapeDtypeStruct((B, LANES), jnp.float32),
        grid=(B // LEN_BLK,),
        in_specs=[pl.BlockSpec((LEN_BLK, L), lambda i: (i, 0))],
        out_specs=pl.BlockSpec((LEN_BLK, LANES), lambda i: (i, 0)),
    )(input_ids)


def _make_sc_kernel():
    mesh = plsc.VectorSubcoreMesh(core_axis_name="c", subcore_axis_name="s")

    @functools.partial(
        pl.kernel,
        mesh=mesh,
        out_type=jax.ShapeDtypeStruct((B, D), jnp.float32),
        scratch_types=[
            pltpu.VMEM((BPW, L), jnp.int32),
            pltpu.VMEM((BPW, LANES), jnp.float32),
            pltpu.VMEM((2, L, D), jnp.float32),
            pltpu.VMEM((BPW, D), jnp.float32),
            pltpu.SemaphoreType.DMA,
            pltpu.SemaphoreType.DMA,
        ],
    )
    def k(ids_hbm, lens_hbm, table_hbm, out_hbm,
          ids_v, lens_v, rows_v, out_v, sem0, sem1):
        wid = lax.axis_index("s") * NC + lax.axis_index("c")
        base = wid * BPW
        pltpu.sync_copy(ids_hbm.at[pl.ds(base, BPW)], ids_v)
        pltpu.sync_copy(lens_hbm.at[pl.ds(base, BPW)], lens_v)

        sems = (sem0, sem1)

        def issue(s, buf):
            for off, n in CHUNKS:
                pltpu.make_async_copy(
                    table_hbm.at[ids_v.at[s, pl.ds(off, n)]],
                    rows_v.at[buf, pl.ds(off, n)],
                    sems[buf],
                ).start()

        def wait(buf):
            for off, n in CHUNKS:
                pltpu.make_async_copy(
                    table_hbm.at[pl.ds(0, n)],
                    rows_v.at[buf, pl.ds(off, n)],
                    sems[buf],
                ).wait()

        def compute(s, buf):
            len_f = lens_v[s, pl.ds(0, LANES)]
            len_i = len_f.astype(jnp.int32)
            zero = jnp.zeros((LANES,), jnp.float32)

            def body(l, acc):
                pred = jnp.full((LANES,), l, jnp.int32) < len_i
                return tuple(
                    acc[kk] + jnp.where(pred, rows_v[buf, l, pl.ds(kk * 16, 16)], zero)
                    for kk in range(DV)
                )

            acc = lax.fori_loop(0, L, body, tuple(zero for _ in range(DV)))
            for kk in range(DV):
                out_v[s, pl.ds(kk * 16, 16)] = acc[kk] / len_f

        issue(0, 0)
        issue(1, 1)

        def pair_body(i, carry):
            s0 = 2 * i
            for buf in range(2):
                s = s0 + buf
                wait(buf)
                compute(s, buf)

                @pl.when(s + 2 < BPW)
                def _():
                    issue(s + 2, buf)

            return carry

        lax.fori_loop(0, BPW // 2, pair_body, 0)
        pltpu.sync_copy(out_v, out_hbm.at[pl.ds(base, BPW)])

    return k


_sc_kernel = _make_sc_kernel()


@jax.jit
def kernel(input_ids, table):
    ids = input_ids.astype(jnp.int32)
    lens = _lengths(ids)
    return _sc_kernel(ids, lens, table)


# speedup vs baseline: 13.8402x; 1.0004x over previous
import functools

import jax
import jax.numpy as jnp
from jax import lax
from jax.experimental import pallas as pl
from jax.experimental.pallas import tpu as pltpu
from jax.experimental.pallas import tpu_sc as plsc

B = 4096
L = 200
D = 128
NC = 2
NS = 16
LANES = 16
NW = NC * NS
BPW = B // NW
DV = D // LANES
LEN_BLK = 512
CHUNKS = ((0, 128), (128, 72))


def _len_body(ids_ref, out_ref):
    lens = jnp.sum((ids_ref[...] != 0).astype(jnp.float32), axis=1, keepdims=True)
    out_ref[...] = jnp.broadcast_to(lens, (LEN_BLK, LANES))


def _lengths(input_ids):
    return pl.pallas_call(
        _len_body,
        out_shape=jax.ShapeDtypeStruct((B, LANES), jnp.float32),
        grid=(B // LEN_BLK,),
        in_specs=[pl.BlockSpec((LEN_BLK, L), lambda i: (i, 0))],
        out_specs=pl.BlockSpec((LEN_BLK, LANES), lambda i: (i, 0)),
    )(input_ids)


def _make_sc_kernel():
    mesh = plsc.VectorSubcoreMesh(core_axis_name="c", subcore_axis_name="s")

    @functools.partial(
        pl.kernel,
        mesh=mesh,
        out_type=jax.ShapeDtypeStruct((B, D), jnp.float32),
        scratch_types=[
            pltpu.VMEM((BPW, L), jnp.int32),
            pltpu.VMEM((BPW, LANES), jnp.float32),
            pltpu.VMEM((2, L, D), jnp.float32),
            pltpu.VMEM((BPW, D), jnp.float32),
            pltpu.SemaphoreType.DMA,
            pltpu.SemaphoreType.DMA,
        ],
    )
    def k(ids_hbm, lens_hbm, table_hbm, out_hbm,
          ids_v, lens_v, rows_v, out_v, sem0, sem1):
        wid = lax.axis_index("s") * NC + lax.axis_index("c")
        base = wid * BPW
        pltpu.sync_copy(ids_hbm.at[pl.ds(base, BPW)], ids_v)
        pltpu.sync_copy(lens_hbm.at[pl.ds(base, BPW)], lens_v)

        sems = (sem0, sem1)

        def issue(s, buf):
            for off, n in CHUNKS:
                pltpu.make_async_copy(
                    table_hbm.at[ids_v.at[s, pl.ds(off, n)]],
                    rows_v.at[buf, pl.ds(off, n)],
                    sems[buf],
                ).start()

        def wait(buf):
            for off, n in CHUNKS:
                pltpu.make_async_copy(
                    table_hbm.at[pl.ds(0, n)],
                    rows_v.at[buf, pl.ds(off, n)],
                    sems[buf],
                ).wait()

        def compute(s, buf):
            len_f = lens_v[s, pl.ds(0, LANES)]
            len_i = len_f.astype(jnp.int32)
            zero = jnp.zeros((LANES,), jnp.float32)

            def body(t, acc):
                l0 = 4 * t
                for dl in range(4):
                    l = l0 + dl
                    pred = jnp.full((LANES,), l, jnp.int32) < len_i
                    acc = tuple(
                        acc[kk]
                        + jnp.where(pred, rows_v[buf, l, pl.ds(kk * 16, 16)], zero)
                        for kk in range(DV)
                    )
                return acc

            acc = lax.fori_loop(0, L // 4, body, tuple(zero for _ in range(DV)))
            for kk in range(DV):
                out_v[s, pl.ds(kk * 16, 16)] = acc[kk] / len_f

        issue(0, 0)
        issue(1, 1)

        def pair_body(i, carry):
            s0 = 2 * i
            for buf in range(2):
                s = s0 + buf
                wait(buf)
                compute(s, buf)

                @pl.when(s + 2 < BPW)
                def _():
                    issue(s + 2, buf)

            return carry

        lax.fori_loop(0, BPW // 2, pair_body, 0)
        pltpu.sync_copy(out_v, out_hbm.at[pl.ds(base, BPW)])

    return k


_sc_kernel = _make_sc_kernel()


@jax.jit
def kernel(input_ids, table):
    ids = input_ids.astype(jnp.int32)
    lens = _lengths(ids)
    return _sc_kernel(ids, lens, table)


# speedup vs baseline: 16.4510x; 1.1886x over previous
import functools

import jax
import jax.numpy as jnp
from jax import lax
from jax.experimental import pallas as pl
from jax.experimental.pallas import tpu as pltpu
from jax.experimental.pallas import tpu_sc as plsc

B = 4096
L = 200
D = 128
VOCAB = 100000
NC = 2
NS = 16
LANES = 16
NW = NC * NS
BPW = B // NW
DV = D // LANES
PREP_BLK = 512
HALves = ((0, 104), (128, 96))
HROWS = 104


LP = 256


def _prep_body(ids_ref, idc_ref, lens_ref):
    x = ids_ref[...]
    lens = jnp.sum((x != 0).astype(jnp.int32), axis=1, keepdims=True)
    lens_ref[...] = jnp.broadcast_to(lens.astype(jnp.float32), (PREP_BLK, LANES))
    pos = lax.broadcasted_iota(jnp.int32, (PREP_BLK, L), 1)
    c = jnp.where(pos < lens, x, 0)
    zpad = jnp.zeros((PREP_BLK, 128 - HROWS), jnp.int32)
    zend = jnp.zeros((PREP_BLK, LP - 128 - (L - HROWS)), jnp.int32)
    idc_ref[...] = jnp.concatenate(
        [c[:, :HROWS], zpad, c[:, HROWS:], zend], axis=1
    )


def _prep(input_ids):
    return pl.pallas_call(
        _prep_body,
        out_shape=[
            jax.ShapeDtypeStruct((B, LP), jnp.int32),
            jax.ShapeDtypeStruct((B, LANES), jnp.float32),
        ],
        grid=(B // PREP_BLK,),
        in_specs=[pl.BlockSpec((PREP_BLK, L), lambda i: (i, 0))],
        out_specs=[
            pl.BlockSpec((PREP_BLK, LP), lambda i: (i, 0)),
            pl.BlockSpec((PREP_BLK, LANES), lambda i: (i, 0)),
        ],
    )(input_ids)


def _make_sc_kernel():
    mesh = plsc.VectorSubcoreMesh(core_axis_name="c", subcore_axis_name="s")

    @functools.partial(
        pl.kernel,
        mesh=mesh,
        out_type=jax.ShapeDtypeStruct((B, D), jnp.float32),
        scratch_types=[
            pltpu.VMEM((BPW, LP), jnp.int32),
            pltpu.VMEM((BPW, LANES), jnp.float32),
            pltpu.VMEM((4, HROWS, D), jnp.float32),
            pltpu.VMEM((8, D), jnp.float32),
            pltpu.VMEM((BPW, D), jnp.float32),
            pltpu.SemaphoreType.DMA,
            pltpu.SemaphoreType.DMA,
            pltpu.SemaphoreType.DMA,
            pltpu.SemaphoreType.DMA,
        ],
    )
    def k(ids_hbm, lens_hbm, table_hbm, out_hbm,
          ids_v, lens_v, rows_v, row0_v, out_v, sem0, sem1, sem2, sem3):
        wid = lax.axis_index("s") * NC + lax.axis_index("c")
        base = wid * BPW
        pltpu.sync_copy(ids_hbm.at[pl.ds(base, BPW)], ids_v)
        pltpu.sync_copy(lens_hbm.at[pl.ds(base, BPW)], lens_v)
        pltpu.sync_copy(table_hbm.at[pl.ds(0, 8)], row0_v)

        sems = (sem0, sem1, sem2, sem3)

        def issue(s, half, slot):
            off, n = HALves[half]
            pltpu.make_async_copy(
                table_hbm.at[ids_v.at[s, pl.ds(off, n)]],
                rows_v.at[slot, pl.ds(0, n)],
                sems[slot],
            ).start()

        def wait(half, slot):
            _, n = HALves[half]
            pltpu.make_async_copy(
                table_hbm.at[pl.ds(0, n)],
                rows_v.at[slot, pl.ds(0, n)],
                sems[slot],
            ).wait()

        def accumulate(slot, n_iters, acc):
            def body(t, acc):
                r0 = 4 * t
                for dr in range(4):
                    r = r0 + dr
                    acc = tuple(
                        acc[kk] + rows_v[slot, r, pl.ds(kk * 16, 16)]
                        for kk in range(DV)
                    )
                return acc

            return lax.fori_loop(0, n_iters, body, acc)

        def finalize(s, acc):
            len_f = lens_v[s, pl.ds(0, LANES)]
            ninv = jnp.full((LANES,), float(L), jnp.float32) - len_f
            for kk in range(DV):
                r0 = row0_v[0, pl.ds(kk * 16, 16)]
                out_v[s, pl.ds(kk * 16, 16)] = (acc[kk] - ninv * r0) / len_f

        zero = jnp.zeros((LANES,), jnp.float32)
        zeros8 = tuple(zero for _ in range(DV))

        for slot in range(4):
            issue(slot // 2, slot % 2, slot)

        def quad_body(i, carry):
            s0 = 2 * i
            for half_pair in range(2):
                s = s0 + half_pair
                slot_a = 2 * half_pair
                slot_b = slot_a + 1
                wait(0, slot_a)
                acc = accumulate(slot_a, HALves[0][1] // 4, zeros8)

                @pl.when(s + 2 < BPW)
                def _():
                    issue(s + 2, 0, slot_a)

                wait(1, slot_b)
                acc = accumulate(slot_b, HALves[1][1] // 4, acc)
                finalize(s, acc)

                @pl.when(s + 2 < BPW)
                def _():
                    issue(s + 2, 1, slot_b)

            return carry

        lax.fori_loop(0, BPW // 2, quad_body, 0)
        pltpu.sync_copy(out_v, out_hbm.at[pl.ds(base, BPW)])

    return k


_sc_kernel = _make_sc_kernel()


@jax.jit
def kernel(input_ids, table):
    ids = input_ids.astype(jnp.int32)
    ids_clean, lens = _prep(ids)
    return _sc_kernel(ids_clean, lens, table)
